# baseline (speedup 1.0000x reference)
import functools

import jax
import jax.numpy as jnp
from jax import lax
from jax.experimental import pallas as pl
from jax.experimental.pallas import tpu as pltpu
from jax.experimental.pallas import tpu_sc as plsc

B, N, E, D = 8, 256, 2048, 128
NN = N * N
R = B * NN
NC, NS = 2, 16
NW = NC * NS
GPC = B // NC
CH = E // NS
RB = 32
CT = 80


def _fill_body(pad_ref, w0_ref, w1_ref, w2_ref, out_ref, comb_ref):
    out_ref[...] = jnp.broadcast_to(pad_ref[...].reshape(1, 1, 1, D), (1, RB, N, D))
    c = lax.broadcasted_iota(jnp.int32, (CT, 8), 0)
    t = lax.broadcasted_iota(jnp.int32, (CT, 8), 1)
    oh0 = ((c >> 4) == t).astype(jnp.float32)
    oh1 = (((c >> 1) & 7) == t).astype(jnp.float32)
    oh2 = ((c & 1) == t).astype(jnp.float32)
    comb = (jnp.dot(oh0, w0_ref[...], preferred_element_type=jnp.float32)
            + jnp.dot(oh1, w1_ref[...], preferred_element_type=jnp.float32)
            + jnp.dot(oh2, w2_ref[...], preferred_element_type=jnp.float32))
    comb_ref[...] = jnp.broadcast_to(comb, (NW, CT, D)).reshape(NW * CT, D)


def _fill(pad2d, w0p, w1p, w2p):
    return pl.pallas_call(
        _fill_body,
        grid=(B, N // RB),
        in_specs=[pl.BlockSpec((1, D), lambda b, r: (0, 0)),
                  pl.BlockSpec((8, D), lambda b, r: (0, 0)),
                  pl.BlockSpec((8, D), lambda b, r: (0, 0)),
                  pl.BlockSpec((8, D), lambda b, r: (0, 0))],
        out_specs=[pl.BlockSpec((1, RB, N, D), lambda b, r: (b, r, 0, 0)),
                   pl.BlockSpec((NW * CT, D), lambda b, r: (0, 0))],
        out_shape=[jax.ShapeDtypeStruct((B, N, N, D), jnp.float32),
                   jax.ShapeDtypeStruct((NW * CT, D), jnp.float32)],
    )(pad2d, w0p, w1p, w2p)


@functools.partial(
    pl.kernel,
    mesh=plsc.VectorSubcoreMesh(core_axis_name="c", subcore_axis_name="s"),
    scratch_types=[
        pltpu.VMEM((5, GPC, CH), jnp.int32),
        pltpu.VMEM((GPC, CH), jnp.int32),
        pltpu.VMEM((GPC, CH), jnp.int32),
        pltpu.VMEM((GPC, CH, D), jnp.float32),
        pltpu.VMEM((GPC * 16, D), jnp.float32),
        pltpu.VMEM((D,), jnp.float32),
        pltpu.VMEM((16,), jnp.int32),
        pltpu.VMEM((GPC * 16,), jnp.int32),
        pltpu.SemaphoreType.DMA,
        pltpu.SemaphoreType.DMA,
        pltpu.SemaphoreType.DMA,
    ],
)
def _sc_scatter(big_ref, comb_ref, edata_ref, sl_ref, dridx_ref,
                edata_v, cidx_v, ridx_v, bond_v, sl_v, slrow_v, dtmp_v, dridx_v,
                lsem, gsem, ssem):
    cid = lax.axis_index("c")
    sid = lax.axis_index("s")
    wid = cid * NS + sid
    ld_e = pltpu.async_copy(edata_ref.at[sid, cid], edata_v, lsem)
    ld_s = pltpu.async_copy(sl_ref.at[0], slrow_v, lsem)
    ld_d = pltpu.async_copy(dridx_ref.at[sid], dtmp_v, lsem)
    ld_e.wait()
    gathers = []
    for j in range(GPC):
        g = cid * GPC + j
        for i in range(CH // 16):
            sl = pl.ds(i * 16, 16)
            src16 = edata_v[0, j, sl]
            dst16 = edata_v[1, j, sl]
            a16 = edata_v[2, j, sl]
            b16 = edata_v[3, j, sl]
            c16 = edata_v[4, j, sl]
            ridx_v[j, sl] = g * NN + src16 * N + dst16
            cidx_v[j, sl] = wid * CT + a16 * 16 + b16 * 2 + c16
        gathers.append(pltpu.async_copy(comb_ref.at[cidx_v.at[j]],
                                        bond_v.at[j], gsem))
    ld_s.wait()
    for k in range(D // 16):
        sk = pl.ds(k * 16, 16)
        v = slrow_v[sk]
        for r in range(GPC * 16):
            sl_v[r, sk] = v
    ld_d.wait()
    base16 = dtmp_v[...]
    for j in range(GPC):
        g = cid * GPC + j
        dridx_v[pl.ds(j * 16, 16)] = base16 + g * NN
    scatters = []
    for j in range(GPC):
        gathers[j].wait()
        scatters.append(pltpu.async_copy(bond_v.at[j], big_ref.at[ridx_v.at[j]], ssem))
    for s in scatters:
        s.wait()
    plsc.subcore_barrier()
    pltpu.async_copy(sl_v, big_ref.at[dridx_v], ssem).wait()


def kernel(edge_index, edge_feat, num_nodes, padding_emb, self_loop, W0, W1, W2):
    ei = edge_index.astype(jnp.int32)
    ef = edge_feat.astype(jnp.int32)
    pad2d = padding_emb.reshape(1, D).astype(jnp.float32)
    sl2d = self_loop.reshape(1, D).astype(jnp.float32)
    w0p = jnp.zeros((8, D), jnp.float32).at[:5, :].set(W0)
    w1p = jnp.zeros((8, D), jnp.float32).at[:6, :].set(W1)
    w2p = jnp.zeros((8, D), jnp.float32).at[:2, :].set(W2)
    filled, comb = _fill(pad2d, w0p, w1p, w2p)
    stacked = jnp.stack([ei[:, 0, :], ei[:, 1, :],
                         ef[:, :, 0], ef[:, :, 1], ef[:, :, 2]])
    edata = stacked.reshape(5, NC, GPC, NS, CH).transpose(3, 1, 0, 2, 4)
    nnm1 = jnp.asarray(num_nodes, jnp.int32) - 1
    dridx = (jnp.minimum(jnp.arange(N, dtype=jnp.int32), nnm1) * (N + 1)).reshape(NS, 16)
    buf = jax.new_ref(filled.reshape(R, D))
    _sc_scatter(buf, comb, edata, sl2d, dridx)
    return buf[...].reshape(B, N, N, D)


# speedup vs baseline: 1.0221x; 1.0221x over previous
import functools

import jax
import jax.numpy as jnp
from jax import lax
from jax.experimental import pallas as pl
from jax.experimental.pallas import tpu as pltpu
from jax.experimental.pallas import tpu_sc as plsc

B, N, E, D = 8, 256, 2048, 128
NN = N * N
R = B * NN
NC, NS = 2, 16
NW = NC * NS
GPC = B // NC
CH = E // NS
RB = 32
CT = 80
CTS = 96


def _fill_body(pad_ref, sl_ref, w0_ref, w1_ref, w2_ref, out_ref, comb_ref):
    out_ref[...] = jnp.broadcast_to(pad_ref[...].reshape(1, 1, 1, D), (1, RB, N, D))
    c = lax.broadcasted_iota(jnp.int32, (CTS, 8), 0)
    t = lax.broadcasted_iota(jnp.int32, (CTS, 8), 1)
    pay = c < CT
    oh0 = (((c >> 4) == t) & pay).astype(jnp.float32)
    oh1 = ((((c >> 1) & 7) == t) & pay).astype(jnp.float32)
    oh2 = (((c & 1) == t) & pay).astype(jnp.float32)
    comb = (jnp.dot(oh0, w0_ref[...], preferred_element_type=jnp.float32)
            + jnp.dot(oh1, w1_ref[...], preferred_element_type=jnp.float32)
            + jnp.dot(oh2, w2_ref[...], preferred_element_type=jnp.float32))
    isdiag = (lax.broadcasted_iota(jnp.int32, (CTS, 1), 0) == CT).astype(jnp.float32)
    comb = comb + isdiag * sl_ref[...]
    comb_ref[...] = jnp.broadcast_to(comb, (NW, CTS, D)).reshape(NW * CTS, D)


def _fill(pad2d, sl2d, w0p, w1p, w2p):
    return pl.pallas_call(
        _fill_body,
        grid=(B, N // RB),
        in_specs=[pl.BlockSpec((1, D), lambda b, r: (0, 0)),
                  pl.BlockSpec((1, D), lambda b, r: (0, 0)),
                  pl.BlockSpec((8, D), lambda b, r: (0, 0)),
                  pl.BlockSpec((8, D), lambda b, r: (0, 0)),
                  pl.BlockSpec((8, D), lambda b, r: (0, 0))],
        out_specs=[pl.BlockSpec((1, RB, N, D), lambda b, r: (b, r, 0, 0)),
                   pl.BlockSpec((NW * CTS, D), lambda b, r: (0, 0))],
        out_shape=[jax.ShapeDtypeStruct((B, N, N, D), jnp.float32),
                   jax.ShapeDtypeStruct((NW * CTS, D), jnp.float32)],
    )(pad2d, sl2d, w0p, w1p, w2p)


@functools.partial(
    pl.kernel,
    mesh=plsc.VectorSubcoreMesh(core_axis_name="c", subcore_axis_name="s"),
    scratch_types=[
        pltpu.VMEM((5, GPC, CH), jnp.int32),
        pltpu.VMEM((GPC, CH), jnp.int32),
        pltpu.VMEM((GPC, CH), jnp.int32),
        pltpu.VMEM((GPC, CH, D), jnp.float32),
        pltpu.VMEM((GPC * 16, D), jnp.float32),
        pltpu.VMEM((D,), jnp.float32),
        pltpu.VMEM((16,), jnp.int32),
        pltpu.VMEM((GPC * 16,), jnp.int32),
        pltpu.SemaphoreType.DMA,
        pltpu.SemaphoreType.DMA,
        pltpu.SemaphoreType.DMA,
    ],
)
def _sc_scatter(big_ref, comb_ref, edata_ref, sl_ref, dridx_ref,
                edata_v, cidx_v, ridx_v, bond_v, sl_v, slrow_v, dtmp_v, dridx_v,
                lsem, gsem, ssem):
    cid = lax.axis_index("c")
    sid = lax.axis_index("s")
    wid = cid * NS + sid
    ld_e = pltpu.async_copy(edata_ref.at[sid, cid], edata_v, lsem)
    ld_s = pltpu.async_copy(sl_ref.at[0], slrow_v, lsem)
    ld_d = pltpu.async_copy(dridx_ref.at[sid], dtmp_v, lsem)
    ld_e.wait()
    gathers = []
    for j in range(GPC):
        g = cid * GPC + j
        for i in range(CH // 16):
            sl = pl.ds(i * 16, 16)
            src16 = edata_v[0, j, sl]
            dst16 = edata_v[1, j, sl]
            a16 = edata_v[2, j, sl]
            b16 = edata_v[3, j, sl]
            c16 = edata_v[4, j, sl]
            ridx_v[j, sl] = g * NN + src16 * N + dst16
            cidx_v[j, sl] = jnp.where(src16 == dst16, wid * CTS + CT,
                                      wid * CTS + a16 * 16 + b16 * 2 + c16)
        gathers.append(pltpu.async_copy(comb_ref.at[cidx_v.at[j]],
                                        bond_v.at[j], gsem))
    ld_s.wait()
    for k in range(D // 16):
        sk = pl.ds(k * 16, 16)
        v = slrow_v[sk]
        for r in range(GPC * 16):
            sl_v[r, sk] = v
    ld_d.wait()
    base16 = dtmp_v[...]
    for j in range(GPC):
        g = cid * GPC + j
        dridx_v[pl.ds(j * 16, 16)] = base16 + g * NN
    scatters = [pltpu.async_copy(sl_v, big_ref.at[dridx_v], ssem)]
    for j in range(GPC):
        gathers[j].wait()
        scatters.append(pltpu.async_copy(bond_v.at[j], big_ref.at[ridx_v.at[j]], ssem))
    for s in scatters:
        s.wait()


def kernel(edge_index, edge_feat, num_nodes, padding_emb, self_loop, W0, W1, W2):
    ei = edge_index.astype(jnp.int32)
    ef = edge_feat.astype(jnp.int32)
    pad2d = padding_emb.reshape(1, D).astype(jnp.float32)
    sl2d = self_loop.reshape(1, D).astype(jnp.float32)
    w0p = jnp.zeros((8, D), jnp.float32).at[:5, :].set(W0)
    w1p = jnp.zeros((8, D), jnp.float32).at[:6, :].set(W1)
    w2p = jnp.zeros((8, D), jnp.float32).at[:2, :].set(W2)
    filled, comb = _fill(pad2d, sl2d, w0p, w1p, w2p)
    stacked = jnp.stack([ei[:, 0, :], ei[:, 1, :],
                         ef[:, :, 0], ef[:, :, 1], ef[:, :, 2]])
    edata = stacked.reshape(5, NC, GPC, NS, CH).transpose(3, 1, 0, 2, 4)
    nnm1 = jnp.asarray(num_nodes, jnp.int32) - 1
    dridx = (jnp.minimum(jnp.arange(N, dtype=jnp.int32), nnm1) * (N + 1)).reshape(NS, 16)
    buf = jax.new_ref(filled.reshape(R, D))
    _sc_scatter(buf, comb, edata, sl2d, dridx)
    return buf[...].reshape(B, N, N, D)


# speedup vs baseline: 1.0252x; 1.0031x over previous
import functools

import jax
import jax.numpy as jnp
from jax import lax
from jax.experimental import pallas as pl
from jax.experimental.pallas import tpu as pltpu
from jax.experimental.pallas import tpu_sc as plsc

B, N, E, D = 8, 256, 2048, 128
NN = N * N
R = B * NN
NC, NS = 2, 16
NW = NC * NS
GPC = B // NC
CH = E // NS
RB = 32
CT = 80
CTS = 96


def _fill_body(pad_ref, sl_ref, w0_ref, w1_ref, w2_ref, out_ref, comb_ref):
    out_ref[...] = jnp.broadcast_to(pad_ref[...], (RB * N, D))
    c = lax.broadcasted_iota(jnp.int32, (CTS, 8), 0)
    t = lax.broadcasted_iota(jnp.int32, (CTS, 8), 1)
    pay = c < CT
    oh0 = (((c >> 4) == t) & pay).astype(jnp.float32)
    oh1 = ((((c >> 1) & 7) == t) & pay).astype(jnp.float32)
    oh2 = (((c & 1) == t) & pay).astype(jnp.float32)
    comb = (jnp.dot(oh0, w0_ref[...], preferred_element_type=jnp.float32)
            + jnp.dot(oh1, w1_ref[...], preferred_element_type=jnp.float32)
            + jnp.dot(oh2, w2_ref[...], preferred_element_type=jnp.float32))
    isdiag = (lax.broadcasted_iota(jnp.int32, (CTS, 1), 0) == CT).astype(jnp.float32)
    comb = comb + isdiag * sl_ref[...]
    comb_ref[...] = jnp.broadcast_to(comb, (NW, CTS, D)).reshape(NW * CTS, D)


def _fill(pad2d, sl2d, w0p, w1p, w2p):
    return pl.pallas_call(
        _fill_body,
        grid=(R // (RB * N),),
        in_specs=[pl.BlockSpec((1, D), lambda i: (0, 0)),
                  pl.BlockSpec((1, D), lambda i: (0, 0)),
                  pl.BlockSpec((8, D), lambda i: (0, 0)),
                  pl.BlockSpec((8, D), lambda i: (0, 0)),
                  pl.BlockSpec((8, D), lambda i: (0, 0))],
        out_specs=[pl.BlockSpec((RB * N, D), lambda i: (i, 0)),
                   pl.BlockSpec((NW * CTS, D), lambda i: (0, 0))],
        out_shape=[jax.ShapeDtypeStruct((R, D), jnp.float32),
                   jax.ShapeDtypeStruct((NW * CTS, D), jnp.float32)],
    )(pad2d, sl2d, w0p, w1p, w2p)


@functools.partial(
    pl.kernel,
    mesh=plsc.VectorSubcoreMesh(core_axis_name="c", subcore_axis_name="s"),
    scratch_types=[
        pltpu.VMEM((5, GPC, CH), jnp.int32),
        pltpu.VMEM((GPC, CH), jnp.int32),
        pltpu.VMEM((GPC, CH), jnp.int32),
        pltpu.VMEM((GPC, CH, D), jnp.float32),
        pltpu.VMEM((GPC * 16, D), jnp.float32),
        pltpu.VMEM((D,), jnp.float32),
        pltpu.VMEM((16,), jnp.int32),
        pltpu.VMEM((GPC * 16,), jnp.int32),
        pltpu.SemaphoreType.DMA,
        pltpu.SemaphoreType.DMA,
        pltpu.SemaphoreType.DMA,
    ],
)
def _sc_scatter(big_ref, comb_ref, edata_ref, sl_ref, dridx_ref,
                edata_v, cidx_v, ridx_v, bond_v, sl_v, slrow_v, dtmp_v, dridx_v,
                lsem, gsem, ssem):
    cid = lax.axis_index("c")
    sid = lax.axis_index("s")
    wid = cid * NS + sid
    ld_e = pltpu.async_copy(edata_ref.at[sid, cid], edata_v, lsem)
    ld_s = pltpu.async_copy(sl_ref.at[0], slrow_v, lsem)
    ld_d = pltpu.async_copy(dridx_ref.at[sid], dtmp_v, lsem)
    ld_e.wait()
    gathers = []
    for j in range(GPC):
        g = cid * GPC + j
        for i in range(CH // 16):
            sl = pl.ds(i * 16, 16)
            src16 = edata_v[0, j, sl]
            dst16 = edata_v[1, j, sl]
            a16 = edata_v[2, j, sl]
            b16 = edata_v[3, j, sl]
            c16 = edata_v[4, j, sl]
            ridx_v[j, sl] = g * NN + src16 * N + dst16
            cidx_v[j, sl] = jnp.where(src16 == dst16, wid * CTS + CT,
                                      wid * CTS + a16 * 16 + b16 * 2 + c16)
        gathers.append(pltpu.async_copy(comb_ref.at[cidx_v.at[j]],
                                        bond_v.at[j], gsem))
    ld_s.wait()
    for k in range(D // 16):
        sk = pl.ds(k * 16, 16)
        v = slrow_v[sk]
        for r in range(GPC * 16):
            sl_v[r, sk] = v
    ld_d.wait()
    base16 = dtmp_v[...]
    for j in range(GPC):
        g = cid * GPC + j
        dridx_v[pl.ds(j * 16, 16)] = base16 + g * NN
    scatters = [pltpu.async_copy(sl_v, big_ref.at[dridx_v], ssem)]
    for j in range(GPC):
        gathers[j].wait()
        scatters.append(pltpu.async_copy(bond_v.at[j], big_ref.at[ridx_v.at[j]], ssem))
    for s in scatters:
        s.wait()


def kernel(edge_index, edge_feat, num_nodes, padding_emb, self_loop, W0, W1, W2):
    ei = edge_index.astype(jnp.int32)
    ef = edge_feat.astype(jnp.int32)
    pad2d = padding_emb.reshape(1, D).astype(jnp.float32)
    sl2d = self_loop.reshape(1, D).astype(jnp.float32)
    w0p = jnp.zeros((8, D), jnp.float32).at[:5, :].set(W0)
    w1p = jnp.zeros((8, D), jnp.float32).at[:6, :].set(W1)
    w2p = jnp.zeros((8, D), jnp.float32).at[:2, :].set(W2)
    filled, comb = _fill(pad2d, sl2d, w0p, w1p, w2p)
    stacked = jnp.stack([ei[:, 0, :], ei[:, 1, :],
                         ef[:, :, 0], ef[:, :, 1], ef[:, :, 2]])
    edata = stacked.reshape(5, NC, GPC, NS, CH).transpose(3, 1, 0, 2, 4)
    nnm1 = jnp.asarray(num_nodes, jnp.int32) - 1
    dridx = (jnp.minimum(jnp.arange(N, dtype=jnp.int32), nnm1) * (N + 1)).reshape(NS, 16)
    buf = jax.new_ref(filled)
    _sc_scatter(buf, comb, edata, sl2d, dridx)
    return buf[...].reshape(B, N, N, D)
